# Initial kernel scaffold; baseline (speedup 1.0000x reference)
import jax
import jax.numpy as jnp
from jax.experimental import pallas as pl


def kernel(x, weight_t_padded):
    raise NotImplementedError("write your pallas kernel here")


# speedup vs baseline: 1.9536x; 1.9536x over previous
import jax
import jax.numpy as jnp
from jax.experimental import pallas as pl
from jax.experimental.pallas import tpu as pltpu

_TM = 1024
_TN = 1024


def _mm_kernel(x_ref, w_ref, o_ref):
    o_ref[...] = jnp.dot(
        x_ref[...], w_ref[...], preferred_element_type=jnp.float32
    )


def kernel(x, weight_t_padded):
    M, K = x.shape
    Kp, N = weight_t_padded.shape
    assert Kp == K and M % _TM == 0 and N % _TN == 0, (M, K, Kp, N)

    xb = x.astype(jnp.bfloat16)
    wb = weight_t_padded.astype(jnp.bfloat16)

    grid = (M // _TM, N // _TN)
    out = pl.pallas_call(
        _mm_kernel,
        out_shape=jax.ShapeDtypeStruct((M, N), jnp.float32),
        grid_spec=pltpu.PrefetchScalarGridSpec(
            num_scalar_prefetch=0,
            grid=grid,
            in_specs=[
                pl.BlockSpec((_TM, K), lambda i, j: (i, 0)),
                pl.BlockSpec((K, _TN), lambda i, j: (0, j)),
            ],
            out_specs=pl.BlockSpec((_TM, _TN), lambda i, j: (i, j)),
        ),
        compiler_params=pltpu.CompilerParams(
            dimension_semantics=("parallel", "parallel"),
            vmem_limit_bytes=60 * 1024 * 1024,
        ),
        cost_estimate=pl.CostEstimate(
            flops=2 * M * K * N,
            transcendentals=0,
            bytes_accessed=(grid[1] * M * K + grid[0] * K * N) * 2 + M * N * 4,
        ),
    )(xb, wb)
    return out


# speedup vs baseline: 2.2157x; 1.1342x over previous
import jax
import jax.numpy as jnp
from jax.experimental import pallas as pl
from jax.experimental.pallas import tpu as pltpu

_TM = 256


def _mm_kernel(x_ref, w_ref, o_ref):
    xb = x_ref[...].astype(jnp.bfloat16)
    o_ref[...] = jnp.dot(xb, w_ref[...], preferred_element_type=jnp.float32)


def kernel(x, weight_t_padded):
    M, K = x.shape
    Kp, N = weight_t_padded.shape
    assert Kp == K and M % _TM == 0, (M, K, Kp, N)

    wb = weight_t_padded.astype(jnp.bfloat16)

    out = pl.pallas_call(
        _mm_kernel,
        out_shape=jax.ShapeDtypeStruct((M, N), jnp.float32),
        grid_spec=pltpu.PrefetchScalarGridSpec(
            num_scalar_prefetch=0,
            grid=(M // _TM,),
            in_specs=[
                pl.BlockSpec((_TM, K), lambda i: (i, 0)),
                pl.BlockSpec((K, N), lambda i: (0, 0)),
            ],
            out_specs=pl.BlockSpec((_TM, N), lambda i: (i, 0)),
        ),
        compiler_params=pltpu.CompilerParams(
            dimension_semantics=("parallel",),
            vmem_limit_bytes=64 * 1024 * 1024,
        ),
        cost_estimate=pl.CostEstimate(
            flops=2 * M * K * N,
            transcendentals=0,
            bytes_accessed=M * K * 4 + K * N * 2 + M * N * 4,
        ),
    )(x, wb)
    return out
